# baseline (speedup 1.0000x reference)
import functools

import jax
import jax.numpy as jnp
from jax import lax
from jax.experimental import pallas as pl
from jax.experimental.pallas import tpu as pltpu
from jax.experimental.pallas import tpu_sc as plsc

N = 10000
E = 320000
D = 128
H = 128
C = 10
G = 64

NC = 2
NS = 16
NW = NC * NS

CH = 128
NCHUNK = 80
HALF = 40
EPAD = NW * NCHUNK * CH
NPAD = 10240
ROWS_PT = NPAD // NS
OUT_PT = 624
TAIL_OFF = NS * OUT_PT
TAIL = N - TAIL_OFF

BLK = 80
NBLK = N // BLK
NBLKP = NPAD // BLK

def _deg_body(dstr, out, dst_v, ones_v, acc_sh):
    cid = lax.axis_index("c")
    sid = lax.axis_index("s")
    wid = sid * NC + cid
    z16 = jnp.zeros((16,), jnp.float32)
    o16 = jnp.ones((16,), jnp.float32)

    def zb(i, _):
        for l in range(H // 16):
            ones_v[i, pl.ds(l * 16, 16)] = z16
        return 0
    lax.fori_loop(0, CH, zb, 0)
    for k in range(ROWS_PT // CH):
        pltpu.sync_copy(ones_v,
                        acc_sh.at[pl.ds(sid * ROWS_PT + k * CH, CH)])

    def ob(i, _):
        for l in range(H // 16):
            ones_v[i, pl.ds(l * 16, 16)] = o16
        return 0
    lax.fori_loop(0, CH, ob, 0)
    pltpu.sync_copy(dstr.at[wid], dst_v)
    plsc.subcore_barrier()

    def chunk(j, _):
        pltpu.sync_copy(ones_v, acc_sh.at[dst_v.at[j]], add=True)
        return 0
    lax.fori_loop(0, NCHUNK, chunk, 0)
    plsc.subcore_barrier()
    pltpu.sync_copy(acc_sh.at[pl.ds(sid * OUT_PT, OUT_PT)],
                    out.at[cid, pl.ds(sid * OUT_PT, OUT_PT)])

    @pl.when(sid == NS - 1)
    def _():
        pltpu.sync_copy(acc_sh.at[pl.ds(TAIL_OFF, TAIL)],
                        out.at[cid, pl.ds(TAIL_OFF, TAIL)])


@functools.lru_cache(maxsize=None)
def _deg_kernel():
    mesh = plsc.VectorSubcoreMesh(core_axis_name="c", subcore_axis_name="s")
    return pl.kernel(
        _deg_body,
        out_type=jax.ShapeDtypeStruct((NC, N, H), jnp.float32),
        mesh=mesh,
        scratch_types=[
            pltpu.VMEM((NCHUNK, CH), jnp.int32),
            pltpu.VMEM((CH, H), jnp.float32),
            pltpu.VMEM_SHARED((NPAD, H), jnp.float32),
        ],
    )


def _mp_body(ts, srcr, dstr, out, src_v, dst_v, rows0, rows1, acc_sh,
             sem0, sem1):
    cid = lax.axis_index("c")
    sid = lax.axis_index("s")
    wid = sid * NC + cid
    z16 = jnp.zeros((16,), jnp.float32)

    def zb(i, _):
        for l in range(H // 16):
            rows0[i, pl.ds(l * 16, 16)] = z16
        return 0
    lax.fori_loop(0, CH, zb, 0)
    for k in range(ROWS_PT // CH):
        pltpu.sync_copy(rows0,
                        acc_sh.at[pl.ds(sid * ROWS_PT + k * CH, CH)])

    pltpu.sync_copy(srcr.at[wid], src_v)
    pltpu.sync_copy(dstr.at[wid], dst_v)
    plsc.subcore_barrier()

    def chunk(j, _):
        pltpu.async_copy(ts.at[src_v.at[j]], rows0, sem0).wait()
        pltpu.sync_copy(rows0, acc_sh.at[dst_v.at[j]], add=True)
        return 0
    lax.fori_loop(0, NCHUNK, chunk, 0)
    plsc.subcore_barrier()
    pltpu.sync_copy(acc_sh.at[pl.ds(sid * OUT_PT, OUT_PT)],
                    out.at[cid, pl.ds(sid * OUT_PT, OUT_PT)])

    @pl.when(sid == NS - 1)
    def _():
        pltpu.sync_copy(acc_sh.at[pl.ds(TAIL_OFF, TAIL)],
                        out.at[cid, pl.ds(TAIL_OFF, TAIL)])


@functools.lru_cache(maxsize=None)
def _mp_kernel():
    mesh = plsc.VectorSubcoreMesh(core_axis_name="c", subcore_axis_name="s")
    return pl.kernel(
        _mp_body,
        out_type=jax.ShapeDtypeStruct((NC, N, H), jnp.float32),
        mesh=mesh,
        scratch_types=[
            pltpu.VMEM((NCHUNK, CH), jnp.int32),
            pltpu.VMEM((NCHUNK, CH), jnp.int32),
            pltpu.VMEM((CH, H), jnp.float32),
            pltpu.VMEM((CH, H), jnp.float32),
            pltpu.VMEM_SHARED((NPAD, H), jnp.float32),
            pltpu.SemaphoreType.DMA,
            pltpu.SemaphoreType.DMA,
        ],
    )


def _ln_relu(h, g, b):
    mu = jnp.mean(h, axis=-1, keepdims=True)
    d = h - mu
    var = jnp.mean(d * d, axis=-1, keepdims=True)
    return jnp.maximum(d * lax.rsqrt(var + 1e-5) * g + b, 0.0)


def _pre_body(x_ref, wp_ref, bp_ref, gp_ref, bep_ref, o_ref):
    h = jnp.dot(x_ref[...], wp_ref[...],
                preferred_element_type=jnp.float32) + bp_ref[...]
    o_ref[...] = _ln_relu(h, gp_ref[...], bep_ref[...])


def _dinv_of(degp_blk):
    return lax.rsqrt(degp_blk[0, :, :1] + degp_blk[1, :, :1] + 1.0)


def _ts_body(h_ref, degp_ref, w_ref, o_ref):
    i = pl.program_id(0)
    dinv = _dinv_of(degp_ref[...])
    t = jnp.dot(h_ref[...], w_ref[...], preferred_element_type=jnp.float32)
    o_ref[...] = jnp.where(i < NBLK, t * dinv, 0.0)


def _mid_body(acc_ref, tsp_ref, degp_ref, b_ref, g_ref, be_ref, w_ref, o_ref):
    i = pl.program_id(0)
    dinv = _dinv_of(degp_ref[...])
    conv = (acc_ref[0] + acc_ref[1] + tsp_ref[...]) * dinv + b_ref[...]
    h = _ln_relu(conv, g_ref[...], be_ref[...])
    t = jnp.dot(h, w_ref[...], preferred_element_type=jnp.float32)
    o_ref[...] = jnp.where(i < NBLK, t * dinv, 0.0)


def _read_body(acc_ref, tsp_ref, degp_ref, b_ref, g_ref, be_ref, bat_ref,
               wpost_ref, o_ref, sums, cnts):
    i = pl.program_id(0)

    @pl.when(i == 0)
    def _():
        sums[...] = jnp.zeros_like(sums)
        cnts[...] = jnp.zeros_like(cnts)

    dinv = _dinv_of(degp_ref[...])
    conv = (acc_ref[0] + acc_ref[1] + tsp_ref[...]) * dinv + b_ref[...]
    h = _ln_relu(conv, g_ref[...], be_ref[...])
    bt = bat_ref[0, 0, :]
    onehot = (lax.broadcasted_iota(jnp.int32, (G, BLK), 0)
              == bt[None, :]).astype(jnp.float32)
    sums[...] += jnp.dot(onehot, h, preferred_element_type=jnp.float32)
    cnts[...] = cnts[...] + jnp.sum(onehot, axis=1, keepdims=True)

    @pl.when(i == NBLK - 1)
    def _():
        emb = sums[...] / jnp.maximum(cnts[...], 1.0)
        logits = jnp.dot(emb, wpost_ref[...],
                         preferred_element_type=jnp.float32)
        col = lax.broadcasted_iota(jnp.int32, (G, H), 1)
        lp = jnp.where(col < C, logits, -1e30)
        m = jnp.max(lp, axis=1, keepdims=True)
        z = lp - m
        lse = jnp.log(jnp.sum(jnp.exp(z), axis=1, keepdims=True))
        o_ref[...] = z - lse


def _row_spec(bs):
    return pl.BlockSpec(bs, lambda i: (i, 0))


def _clamped_row_spec(bs):
    return pl.BlockSpec(bs, lambda i: (jnp.minimum(i, NBLK - 1), 0))


def _full_spec(bs):
    return pl.BlockSpec(bs, lambda i: tuple(0 for _ in bs))


def kernel(x, edge_index, batch, W_pre, b_pre, g_pre, be_pre, W1, b1, g1,
           be1, W2, b2, g2, be2, W_post):
    f32 = jnp.float32
    ei = jnp.concatenate(
        [edge_index, jnp.full((2, EPAD - E), N, jnp.int32)], axis=1)
    srcr = ei[0].reshape(NW, NCHUNK, CH)
    dstr = ei[1].reshape(NW, NCHUNK, CH)
    batch3 = batch.reshape(NBLK, 1, BLK)
    b_pre2, g_pre2, be_pre2 = (b_pre.reshape(1, H), g_pre.reshape(1, H),
                               be_pre.reshape(1, H))
    b12, g12, be12 = b1.reshape(1, H), g1.reshape(1, H), be1.reshape(1, H)
    b22, g22, be22 = b2.reshape(1, H), g2.reshape(1, H), be2.reshape(1, H)
    wpost_pad = jnp.pad(W_post, ((0, 0), (0, H - C)))

    degp = _deg_kernel()(dstr)

    h0 = pl.pallas_call(
        _pre_body,
        grid=(NBLK,),
        in_specs=[_row_spec((BLK, D)), _full_spec((D, H)), _full_spec((1, H)),
                  _full_spec((1, H)), _full_spec((1, H))],
        out_specs=_row_spec((BLK, H)),
        out_shape=jax.ShapeDtypeStruct((N, H), f32),
    )(x, W_pre, b_pre2, g_pre2, be_pre2)

    acc_spec = pl.BlockSpec((NC, BLK, H),
                            lambda i: (0, jnp.minimum(i, NBLK - 1), 0))
    deg_spec = acc_spec

    ts1 = pl.pallas_call(
        _ts_body,
        grid=(NBLKP,),
        in_specs=[_clamped_row_spec((BLK, H)), deg_spec, _full_spec((H, H))],
        out_specs=_row_spec((BLK, H)),
        out_shape=jax.ShapeDtypeStruct((NPAD, H), f32),
    )(h0, degp, W1)

    accs1 = _mp_kernel()(ts1, srcr, dstr)

    ts2 = pl.pallas_call(
        _mid_body,
        grid=(NBLKP,),
        in_specs=[acc_spec, _row_spec((BLK, H)), deg_spec,
                  _full_spec((1, H)), _full_spec((1, H)), _full_spec((1, H)),
                  _full_spec((H, H))],
        out_specs=_row_spec((BLK, H)),
        out_shape=jax.ShapeDtypeStruct((NPAD, H), f32),
    )(accs1, ts1, degp, b12, g12, be12, W2)

    accs2 = _mp_kernel()(ts2, srcr, dstr)

    logits = pl.pallas_call(
        _read_body,
        grid=(NBLK,),
        in_specs=[acc_spec, _row_spec((BLK, H)), deg_spec,
                  _full_spec((1, H)), _full_spec((1, H)), _full_spec((1, H)),
                  pl.BlockSpec((1, 1, BLK), lambda i: (i, 0, 0)),
                  _full_spec((H, H))],
        out_specs=_full_spec((G, H)),
        out_shape=jax.ShapeDtypeStruct((G, H), f32),
        scratch_shapes=[pltpu.VMEM((G, H), f32), pltpu.VMEM((G, H), f32)],
    )(accs2, ts2, degp, b22, g22, be22, batch3, wpost_pad)

    return logits[:, :C]


# speedup vs baseline: 1.8809x; 1.8809x over previous
import functools

import jax
import jax.numpy as jnp
from jax import lax
from jax.experimental import pallas as pl
from jax.experimental.pallas import tpu as pltpu
from jax.experimental.pallas import tpu_sc as plsc

N = 10000
E = 320000
D = 128
H = 128
C = 10
G = 64

NC = 2
NS = 16
NW = NC * NS

CH = 128
NCHUNK = 80
HALF = 40
EPAD = NW * NCHUNK * CH
NPAD = 10240
ROWS_PT = NPAD // NS
OUT_PT = 624
TAIL_OFF = NS * OUT_PT
TAIL = N - TAIL_OFF

BLK = 80
NBLK = N // BLK
NBLKP = NPAD // BLK

def _deg_body(dstr, out, dst_v, ones_v, acc_sh):
    cid = lax.axis_index("c")
    sid = lax.axis_index("s")
    wid = sid * NC + cid
    z16 = jnp.zeros((16,), jnp.float32)
    o16 = jnp.ones((16,), jnp.float32)

    def zb(i, _):
        for l in range(H // 16):
            ones_v[i, pl.ds(l * 16, 16)] = z16
        return 0
    lax.fori_loop(0, CH, zb, 0)
    for k in range(ROWS_PT // CH):
        pltpu.sync_copy(ones_v,
                        acc_sh.at[pl.ds(sid * ROWS_PT + k * CH, CH)])

    def ob(i, _):
        for l in range(H // 16):
            ones_v[i, pl.ds(l * 16, 16)] = o16
        return 0
    lax.fori_loop(0, CH, ob, 0)
    pltpu.sync_copy(dstr.at[wid], dst_v)
    plsc.subcore_barrier()

    def chunk(j, _):
        pltpu.sync_copy(ones_v, acc_sh.at[dst_v.at[j]], add=True)
        return 0
    lax.fori_loop(0, NCHUNK, chunk, 0)
    plsc.subcore_barrier()
    pltpu.sync_copy(acc_sh.at[pl.ds(sid * OUT_PT, OUT_PT)],
                    out.at[cid, pl.ds(sid * OUT_PT, OUT_PT)])

    @pl.when(sid == NS - 1)
    def _():
        pltpu.sync_copy(acc_sh.at[pl.ds(TAIL_OFF, TAIL)],
                        out.at[cid, pl.ds(TAIL_OFF, TAIL)])


@functools.lru_cache(maxsize=None)
def _deg_kernel():
    mesh = plsc.VectorSubcoreMesh(core_axis_name="c", subcore_axis_name="s")
    return pl.kernel(
        _deg_body,
        out_type=jax.ShapeDtypeStruct((NC, N, H), jnp.float32),
        mesh=mesh,
        scratch_types=[
            pltpu.VMEM((NCHUNK, CH), jnp.int32),
            pltpu.VMEM((CH, H), jnp.float32),
            pltpu.VMEM_SHARED((NPAD, H), jnp.float32),
        ],
    )


def _mp_body(ts, srcr, dstr, out, src_v, dst_v, rows0, rows1, acc_sh,
             sem0, sem1):
    cid = lax.axis_index("c")
    sid = lax.axis_index("s")
    wid = sid * NC + cid
    z16 = jnp.zeros((16,), jnp.float32)

    def zb(i, _):
        for l in range(H // 16):
            rows0[i, pl.ds(l * 16, 16)] = z16
        return 0
    lax.fori_loop(0, CH, zb, 0)
    for k in range(ROWS_PT // CH):
        pltpu.sync_copy(rows0,
                        acc_sh.at[pl.ds(sid * ROWS_PT + k * CH, CH)])

    pltpu.sync_copy(srcr.at[wid], src_v)
    pltpu.sync_copy(dstr.at[wid], dst_v)
    plsc.subcore_barrier()

    def chunk(j, _):
        pltpu.async_copy(ts.at[src_v.at[j]], rows0, sem0).wait()
        pltpu.sync_copy(rows0, acc_sh.at[dst_v.at[j]], add=True)
        return 0
    lax.fori_loop(0, NCHUNK, chunk, 0)
    plsc.subcore_barrier()
    pltpu.sync_copy(acc_sh.at[pl.ds(sid * OUT_PT, OUT_PT)],
                    out.at[cid, pl.ds(sid * OUT_PT, OUT_PT)])

    @pl.when(sid == NS - 1)
    def _():
        pltpu.sync_copy(acc_sh.at[pl.ds(TAIL_OFF, TAIL)],
                        out.at[cid, pl.ds(TAIL_OFF, TAIL)])


@functools.lru_cache(maxsize=None)
def _mp_kernel():
    mesh = plsc.VectorSubcoreMesh(core_axis_name="c", subcore_axis_name="s")
    return pl.kernel(
        _mp_body,
        out_type=jax.ShapeDtypeStruct((NC, N, H), jnp.float32),
        mesh=mesh,
        scratch_types=[
            pltpu.VMEM((NCHUNK, CH), jnp.int32),
            pltpu.VMEM((NCHUNK, CH), jnp.int32),
            pltpu.VMEM((CH, H), jnp.float32),
            pltpu.VMEM((CH, H), jnp.float32),
            pltpu.VMEM_SHARED((NPAD, H), jnp.float32),
            pltpu.SemaphoreType.DMA,
            pltpu.SemaphoreType.DMA,
        ],
    )


def _ln_relu(h, g, b):
    mu = jnp.mean(h, axis=-1, keepdims=True)
    d = h - mu
    var = jnp.mean(d * d, axis=-1, keepdims=True)
    return jnp.maximum(d * lax.rsqrt(var + 1e-5) * g + b, 0.0)


def _pre_body(x_ref, wp_ref, bp_ref, gp_ref, bep_ref, o_ref):
    h = jnp.dot(x_ref[...], wp_ref[...],
                preferred_element_type=jnp.float32) + bp_ref[...]
    o_ref[...] = _ln_relu(h, gp_ref[...], bep_ref[...])


def _dinv_of(degp_blk):
    return lax.rsqrt(degp_blk[0, :, :1] + degp_blk[1, :, :1] + 1.0)


def _ts_body(h_ref, degp_ref, w_ref, o_ref):
    i = pl.program_id(0)
    dinv = _dinv_of(degp_ref[...])
    t = jnp.dot(h_ref[...], w_ref[...], preferred_element_type=jnp.float32)
    o_ref[...] = jnp.where(i < NBLK, t * dinv, 0.0)


def _mid_body(acc_ref, tsp_ref, degp_ref, b_ref, g_ref, be_ref, w_ref, o_ref):
    i = pl.program_id(0)
    dinv = _dinv_of(degp_ref[...])
    conv = (acc_ref[0] + acc_ref[1] + tsp_ref[...]) * dinv + b_ref[...]
    h = _ln_relu(conv, g_ref[...], be_ref[...])
    t = jnp.dot(h, w_ref[...], preferred_element_type=jnp.float32)
    o_ref[...] = jnp.where(i < NBLK, t * dinv, 0.0)


def _read_body(acc_ref, tsp_ref, degp_ref, b_ref, g_ref, be_ref, bat_ref,
               wpost_ref, o_ref, sums, cnts):
    i = pl.program_id(0)

    @pl.when(i == 0)
    def _():
        sums[...] = jnp.zeros_like(sums)
        cnts[...] = jnp.zeros_like(cnts)

    dinv = _dinv_of(degp_ref[...])
    conv = (acc_ref[0] + acc_ref[1] + tsp_ref[...]) * dinv + b_ref[...]
    h = _ln_relu(conv, g_ref[...], be_ref[...])
    bt = bat_ref[0, 0, :]
    onehot = (lax.broadcasted_iota(jnp.int32, (G, BLK), 0)
              == bt[None, :]).astype(jnp.float32)
    sums[...] += jnp.dot(onehot, h, preferred_element_type=jnp.float32)
    cnts[...] = cnts[...] + jnp.sum(onehot, axis=1, keepdims=True)

    @pl.when(i == NBLK - 1)
    def _():
        emb = sums[...] / jnp.maximum(cnts[...], 1.0)
        logits = jnp.dot(emb, wpost_ref[...],
                         preferred_element_type=jnp.float32)
        col = lax.broadcasted_iota(jnp.int32, (G, H), 1)
        lp = jnp.where(col < C, logits, -1e30)
        m = jnp.max(lp, axis=1, keepdims=True)
        z = lp - m
        lse = jnp.log(jnp.sum(jnp.exp(z), axis=1, keepdims=True))
        o_ref[...] = z - lse


def _row_spec(bs):
    return pl.BlockSpec(bs, lambda i: (i, 0))


def _clamped_row_spec(bs):
    return pl.BlockSpec(bs, lambda i: (jnp.minimum(i, NBLK - 1), 0))


def _full_spec(bs):
    return pl.BlockSpec(bs, lambda i: tuple(0 for _ in bs))


def kernel(x, edge_index, batch, W_pre, b_pre, g_pre, be_pre, W1, b1, g1,
           be1, W2, b2, g2, be2, W_post):
    f32 = jnp.float32
    pad_idx = N + jnp.arange(EPAD - E, dtype=jnp.int32) % (NPAD - N)
    ei = jnp.concatenate(
        [edge_index, jnp.stack([pad_idx, pad_idx])], axis=1)
    srcr = ei[0].reshape(NW, NCHUNK, CH)
    dstr = ei[1].reshape(NW, NCHUNK, CH)
    batch3 = batch.reshape(NBLK, 1, BLK)
    b_pre2, g_pre2, be_pre2 = (b_pre.reshape(1, H), g_pre.reshape(1, H),
                               be_pre.reshape(1, H))
    b12, g12, be12 = b1.reshape(1, H), g1.reshape(1, H), be1.reshape(1, H)
    b22, g22, be22 = b2.reshape(1, H), g2.reshape(1, H), be2.reshape(1, H)
    wpost_pad = jnp.pad(W_post, ((0, 0), (0, H - C)))

    degp = _deg_kernel()(dstr)

    h0 = pl.pallas_call(
        _pre_body,
        grid=(NBLK,),
        in_specs=[_row_spec((BLK, D)), _full_spec((D, H)), _full_spec((1, H)),
                  _full_spec((1, H)), _full_spec((1, H))],
        out_specs=_row_spec((BLK, H)),
        out_shape=jax.ShapeDtypeStruct((N, H), f32),
    )(x, W_pre, b_pre2, g_pre2, be_pre2)

    acc_spec = pl.BlockSpec((NC, BLK, H),
                            lambda i: (0, jnp.minimum(i, NBLK - 1), 0))
    deg_spec = acc_spec

    ts1 = pl.pallas_call(
        _ts_body,
        grid=(NBLKP,),
        in_specs=[_clamped_row_spec((BLK, H)), deg_spec, _full_spec((H, H))],
        out_specs=_row_spec((BLK, H)),
        out_shape=jax.ShapeDtypeStruct((NPAD, H), f32),
    )(h0, degp, W1)

    accs1 = _mp_kernel()(ts1, srcr, dstr)

    ts2 = pl.pallas_call(
        _mid_body,
        grid=(NBLKP,),
        in_specs=[acc_spec, _row_spec((BLK, H)), deg_spec,
                  _full_spec((1, H)), _full_spec((1, H)), _full_spec((1, H)),
                  _full_spec((H, H))],
        out_specs=_row_spec((BLK, H)),
        out_shape=jax.ShapeDtypeStruct((NPAD, H), f32),
    )(accs1, ts1, degp, b12, g12, be12, W2)

    accs2 = _mp_kernel()(ts2, srcr, dstr)

    logits = pl.pallas_call(
        _read_body,
        grid=(NBLK,),
        in_specs=[acc_spec, _row_spec((BLK, H)), deg_spec,
                  _full_spec((1, H)), _full_spec((1, H)), _full_spec((1, H)),
                  pl.BlockSpec((1, 1, BLK), lambda i: (i, 0, 0)),
                  _full_spec((H, H))],
        out_specs=_full_spec((G, H)),
        out_shape=jax.ShapeDtypeStruct((G, H), f32),
        scratch_shapes=[pltpu.VMEM((G, H), f32), pltpu.VMEM((G, H), f32)],
    )(accs2, ts2, degp, b22, g22, be22, batch3, wpost_pad)

    return logits[:, :C]


# speedup vs baseline: 2.1599x; 1.1484x over previous
import functools

import jax
import jax.numpy as jnp
from jax import lax
from jax.experimental import pallas as pl
from jax.experimental.pallas import tpu as pltpu
from jax.experimental.pallas import tpu_sc as plsc

N = 10000
E = 320000
D = 128
H = 128
C = 10
G = 64

NC = 2
NS = 16
NW = NC * NS

CH = 128
NCHUNK = 80
HALF = 40
EPAD = NW * NCHUNK * CH
NPAD = 10240
ROWS_PT = NPAD // NS
OUT_PT = 624
TAIL_OFF = NS * OUT_PT
TAIL = N - TAIL_OFF

BLK = 80
NBLK = N // BLK
NBLKP = NPAD // BLK

def _deg_body(dstr, out, dst_v, ones_v, acc_sh):
    cid = lax.axis_index("c")
    sid = lax.axis_index("s")
    wid = sid * NC + cid
    z16 = jnp.zeros((16,), jnp.float32)
    o16 = jnp.ones((16,), jnp.float32)

    def zb(i, _):
        for l in range(H // 16):
            ones_v[i, pl.ds(l * 16, 16)] = z16
        return 0
    lax.fori_loop(0, CH, zb, 0)
    for k in range(ROWS_PT // CH):
        pltpu.sync_copy(ones_v,
                        acc_sh.at[pl.ds(sid * ROWS_PT + k * CH, CH)])

    def ob(i, _):
        for l in range(H // 16):
            ones_v[i, pl.ds(l * 16, 16)] = o16
        return 0
    lax.fori_loop(0, CH, ob, 0)
    pltpu.sync_copy(dstr.at[wid], dst_v)
    plsc.subcore_barrier()

    def chunk(j, _):
        pltpu.sync_copy(ones_v, acc_sh.at[dst_v.at[j]], add=True)
        return 0
    lax.fori_loop(0, NCHUNK, chunk, 0)
    plsc.subcore_barrier()
    pltpu.sync_copy(acc_sh.at[pl.ds(sid * OUT_PT, OUT_PT)],
                    out.at[cid, pl.ds(sid * OUT_PT, OUT_PT)])

    @pl.when(sid == NS - 1)
    def _():
        pltpu.sync_copy(acc_sh.at[pl.ds(TAIL_OFF, TAIL)],
                        out.at[cid, pl.ds(TAIL_OFF, TAIL)])


@functools.lru_cache(maxsize=None)
def _deg_kernel():
    mesh = plsc.VectorSubcoreMesh(core_axis_name="c", subcore_axis_name="s")
    return pl.kernel(
        _deg_body,
        out_type=jax.ShapeDtypeStruct((NC, N, H), jnp.float32),
        mesh=mesh,
        scratch_types=[
            pltpu.VMEM((NCHUNK, CH), jnp.int32),
            pltpu.VMEM((CH, H), jnp.float32),
            pltpu.VMEM_SHARED((NPAD, H), jnp.float32),
        ],
    )


def _mp_body(ts, srcr, dstr, out, src_v, dst_v, rows0, rows1, acc_sh,
             sem0, sem1):
    cid = lax.axis_index("c")
    sid = lax.axis_index("s")
    wid = sid * NC + cid
    z16 = jnp.zeros((16,), jnp.float32)

    def zb(i, _):
        for l in range(H // 16):
            rows0[i, pl.ds(l * 16, 16)] = z16
        return 0
    lax.fori_loop(0, CH, zb, 0)
    for k in range(ROWS_PT // CH):
        pltpu.sync_copy(rows0,
                        acc_sh.at[pl.ds(sid * ROWS_PT + k * CH, CH)])

    pltpu.sync_copy(dstr.at[wid], dst_v)
    for h in range(NCHUNK // HALF):
        pltpu.sync_copy(srcr.at[wid, pl.ds(h * HALF, HALF)], src_v)
        if h == 0:
            plsc.subcore_barrier()
        pltpu.async_copy(ts.at[src_v.at[0]], rows0, sem0)

        def pair(j, _):
            c = h * HALF + 2 * j
            pltpu.make_async_copy(ts.at[src_v.at[2 * j]], rows0, sem0).wait()
            pltpu.async_copy(ts.at[src_v.at[2 * j + 1]], rows1, sem1)
            pltpu.sync_copy(rows0, acc_sh.at[dst_v.at[c]], add=True)
            pltpu.make_async_copy(ts.at[src_v.at[2 * j + 1]], rows1,
                                  sem1).wait()

            @pl.when(j < HALF // 2 - 1)
            def _():
                pltpu.async_copy(ts.at[src_v.at[2 * j + 2]], rows0, sem0)
            pltpu.sync_copy(rows1, acc_sh.at[dst_v.at[c + 1]], add=True)
            return 0
        lax.fori_loop(0, HALF // 2, pair, 0)
    plsc.subcore_barrier()
    pltpu.sync_copy(acc_sh.at[pl.ds(sid * OUT_PT, OUT_PT)],
                    out.at[cid, pl.ds(sid * OUT_PT, OUT_PT)])

    @pl.when(sid == NS - 1)
    def _():
        pltpu.sync_copy(acc_sh.at[pl.ds(TAIL_OFF, TAIL)],
                        out.at[cid, pl.ds(TAIL_OFF, TAIL)])


@functools.lru_cache(maxsize=None)
def _mp_kernel():
    mesh = plsc.VectorSubcoreMesh(core_axis_name="c", subcore_axis_name="s")
    return pl.kernel(
        _mp_body,
        out_type=jax.ShapeDtypeStruct((NC, N, H), jnp.float32),
        mesh=mesh,
        scratch_types=[
            pltpu.VMEM((HALF, CH), jnp.int32),
            pltpu.VMEM((NCHUNK, CH), jnp.int32),
            pltpu.VMEM((CH, H), jnp.float32),
            pltpu.VMEM((CH, H), jnp.float32),
            pltpu.VMEM_SHARED((NPAD, H), jnp.float32),
            pltpu.SemaphoreType.DMA,
            pltpu.SemaphoreType.DMA,
        ],
    )


def _ln_relu(h, g, b):
    mu = jnp.mean(h, axis=-1, keepdims=True)
    d = h - mu
    var = jnp.mean(d * d, axis=-1, keepdims=True)
    return jnp.maximum(d * lax.rsqrt(var + 1e-5) * g + b, 0.0)


def _pre_body(x_ref, wp_ref, bp_ref, gp_ref, bep_ref, o_ref):
    h = jnp.dot(x_ref[...], wp_ref[...],
                preferred_element_type=jnp.float32) + bp_ref[...]
    o_ref[...] = _ln_relu(h, gp_ref[...], bep_ref[...])


def _dinv_of(degp_blk):
    return lax.rsqrt(degp_blk[0, :, :1] + degp_blk[1, :, :1] + 1.0)


def _ts_body(h_ref, degp_ref, w_ref, o_ref):
    i = pl.program_id(0)
    dinv = _dinv_of(degp_ref[...])
    t = jnp.dot(h_ref[...], w_ref[...], preferred_element_type=jnp.float32)
    o_ref[...] = jnp.where(i < NBLK, t * dinv, 0.0)


def _mid_body(acc_ref, tsp_ref, degp_ref, b_ref, g_ref, be_ref, w_ref, o_ref):
    i = pl.program_id(0)
    dinv = _dinv_of(degp_ref[...])
    conv = (acc_ref[0] + acc_ref[1] + tsp_ref[...]) * dinv + b_ref[...]
    h = _ln_relu(conv, g_ref[...], be_ref[...])
    t = jnp.dot(h, w_ref[...], preferred_element_type=jnp.float32)
    o_ref[...] = jnp.where(i < NBLK, t * dinv, 0.0)


def _read_body(acc_ref, tsp_ref, degp_ref, b_ref, g_ref, be_ref, bat_ref,
               wpost_ref, o_ref, sums, cnts):
    i = pl.program_id(0)

    @pl.when(i == 0)
    def _():
        sums[...] = jnp.zeros_like(sums)
        cnts[...] = jnp.zeros_like(cnts)

    dinv = _dinv_of(degp_ref[...])
    conv = (acc_ref[0] + acc_ref[1] + tsp_ref[...]) * dinv + b_ref[...]
    h = _ln_relu(conv, g_ref[...], be_ref[...])
    bt = bat_ref[0, 0, :]
    onehot = (lax.broadcasted_iota(jnp.int32, (G, BLK), 0)
              == bt[None, :]).astype(jnp.float32)
    sums[...] += jnp.dot(onehot, h, preferred_element_type=jnp.float32)
    cnts[...] = cnts[...] + jnp.sum(onehot, axis=1, keepdims=True)

    @pl.when(i == NBLK - 1)
    def _():
        emb = sums[...] / jnp.maximum(cnts[...], 1.0)
        logits = jnp.dot(emb, wpost_ref[...],
                         preferred_element_type=jnp.float32)
        col = lax.broadcasted_iota(jnp.int32, (G, H), 1)
        lp = jnp.where(col < C, logits, -1e30)
        m = jnp.max(lp, axis=1, keepdims=True)
        z = lp - m
        lse = jnp.log(jnp.sum(jnp.exp(z), axis=1, keepdims=True))
        o_ref[...] = z - lse


def _row_spec(bs):
    return pl.BlockSpec(bs, lambda i: (i, 0))


def _clamped_row_spec(bs):
    return pl.BlockSpec(bs, lambda i: (jnp.minimum(i, NBLK - 1), 0))


def _full_spec(bs):
    return pl.BlockSpec(bs, lambda i: tuple(0 for _ in bs))


def kernel(x, edge_index, batch, W_pre, b_pre, g_pre, be_pre, W1, b1, g1,
           be1, W2, b2, g2, be2, W_post):
    f32 = jnp.float32
    pad_idx = N + jnp.arange(EPAD - E, dtype=jnp.int32) % (NPAD - N)
    ei = jnp.concatenate(
        [edge_index, jnp.stack([pad_idx, pad_idx])], axis=1)
    srcr = ei[0].reshape(NW, NCHUNK, CH)
    dstr = ei[1].reshape(NW, NCHUNK, CH)
    batch3 = batch.reshape(NBLK, 1, BLK)
    b_pre2, g_pre2, be_pre2 = (b_pre.reshape(1, H), g_pre.reshape(1, H),
                               be_pre.reshape(1, H))
    b12, g12, be12 = b1.reshape(1, H), g1.reshape(1, H), be1.reshape(1, H)
    b22, g22, be22 = b2.reshape(1, H), g2.reshape(1, H), be2.reshape(1, H)
    wpost_pad = jnp.pad(W_post, ((0, 0), (0, H - C)))

    degp = _deg_kernel()(dstr)

    h0 = pl.pallas_call(
        _pre_body,
        grid=(NBLK,),
        in_specs=[_row_spec((BLK, D)), _full_spec((D, H)), _full_spec((1, H)),
                  _full_spec((1, H)), _full_spec((1, H))],
        out_specs=_row_spec((BLK, H)),
        out_shape=jax.ShapeDtypeStruct((N, H), f32),
    )(x, W_pre, b_pre2, g_pre2, be_pre2)

    acc_spec = pl.BlockSpec((NC, BLK, H),
                            lambda i: (0, jnp.minimum(i, NBLK - 1), 0))
    deg_spec = acc_spec

    ts1 = pl.pallas_call(
        _ts_body,
        grid=(NBLKP,),
        in_specs=[_clamped_row_spec((BLK, H)), deg_spec, _full_spec((H, H))],
        out_specs=_row_spec((BLK, H)),
        out_shape=jax.ShapeDtypeStruct((NPAD, H), f32),
    )(h0, degp, W1)

    accs1 = _mp_kernel()(ts1, srcr, dstr)

    ts2 = pl.pallas_call(
        _mid_body,
        grid=(NBLKP,),
        in_specs=[acc_spec, _row_spec((BLK, H)), deg_spec,
                  _full_spec((1, H)), _full_spec((1, H)), _full_spec((1, H)),
                  _full_spec((H, H))],
        out_specs=_row_spec((BLK, H)),
        out_shape=jax.ShapeDtypeStruct((NPAD, H), f32),
    )(accs1, ts1, degp, b12, g12, be12, W2)

    accs2 = _mp_kernel()(ts2, srcr, dstr)

    logits = pl.pallas_call(
        _read_body,
        grid=(NBLK,),
        in_specs=[acc_spec, _row_spec((BLK, H)), deg_spec,
                  _full_spec((1, H)), _full_spec((1, H)), _full_spec((1, H)),
                  pl.BlockSpec((1, 1, BLK), lambda i: (i, 0, 0)),
                  _full_spec((H, H))],
        out_specs=_full_spec((G, H)),
        out_shape=jax.ShapeDtypeStruct((G, H), f32),
        scratch_shapes=[pltpu.VMEM((G, H), f32), pltpu.VMEM((G, H), f32)],
    )(accs2, ts2, degp, b22, g22, be22, batch3, wpost_pad)

    return logits[:, :C]


# speedup vs baseline: 3.3771x; 1.5635x over previous
import functools

import jax
import jax.numpy as jnp
from jax import lax
from jax.experimental import pallas as pl
from jax.experimental.pallas import tpu as pltpu
from jax.experimental.pallas import tpu_sc as plsc

N = 10000
E = 320000
D = 128
H = 128
C = 10
G = 64

NC = 2
NS = 16
NW = NC * NS

CH = 128
NCHUNK = 80
HALF = 40
EPAD = NW * NCHUNK * CH
NPAD = 10240
ROWS_PT = NPAD // NS
OUT_PT = 624
TAIL_OFF = NS * OUT_PT
TAIL = N - TAIL_OFF

BLK = 2000
NBLK = N // BLK

def _deg_body(dstr, out, dst_v, ones_v, acc_sh):
    cid = lax.axis_index("c")
    sid = lax.axis_index("s")
    wid = sid * NC + cid
    z16 = jnp.zeros((16,), jnp.float32)
    o16 = jnp.ones((16,), jnp.float32)

    def zb(i, _):
        for l in range(H // 16):
            ones_v[i, pl.ds(l * 16, 16)] = z16
        return 0
    lax.fori_loop(0, CH, zb, 0)
    for k in range(ROWS_PT // CH):
        pltpu.sync_copy(ones_v,
                        acc_sh.at[pl.ds(sid * ROWS_PT + k * CH, CH)])

    def ob(i, _):
        for l in range(H // 16):
            ones_v[i, pl.ds(l * 16, 16)] = o16
        return 0
    lax.fori_loop(0, CH, ob, 0)
    pltpu.sync_copy(dstr.at[wid], dst_v)
    plsc.subcore_barrier()

    def chunk(j, _):
        pltpu.sync_copy(ones_v, acc_sh.at[dst_v.at[j]], add=True)
        return 0
    lax.fori_loop(0, NCHUNK, chunk, 0)
    plsc.subcore_barrier()
    pltpu.sync_copy(acc_sh.at[pl.ds(sid * OUT_PT, OUT_PT)],
                    out.at[cid, pl.ds(sid * OUT_PT, OUT_PT)])

    @pl.when(sid == NS - 1)
    def _():
        pltpu.sync_copy(acc_sh.at[pl.ds(TAIL_OFF, TAIL)],
                        out.at[cid, pl.ds(TAIL_OFF, TAIL)])


@functools.lru_cache(maxsize=None)
def _deg_kernel():
    mesh = plsc.VectorSubcoreMesh(core_axis_name="c", subcore_axis_name="s")
    return pl.kernel(
        _deg_body,
        out_type=jax.ShapeDtypeStruct((NC, N, H), jnp.float32),
        mesh=mesh,
        scratch_types=[
            pltpu.VMEM((NCHUNK, CH), jnp.int32),
            pltpu.VMEM((CH, H), jnp.float32),
            pltpu.VMEM_SHARED((NPAD, H), jnp.float32),
        ],
    )


def _mp_body(ts, srcr, dstr, out, src_v, dst_v, rows0, rows1, acc_sh,
             sem0, sem1):
    cid = lax.axis_index("c")
    sid = lax.axis_index("s")
    wid = sid * NC + cid
    z16 = jnp.zeros((16,), jnp.float32)

    def zb(i, _):
        for l in range(H // 16):
            rows0[i, pl.ds(l * 16, 16)] = z16
        return 0
    lax.fori_loop(0, CH, zb, 0)
    for k in range(ROWS_PT // CH):
        pltpu.sync_copy(rows0,
                        acc_sh.at[pl.ds(sid * ROWS_PT + k * CH, CH)])

    pltpu.sync_copy(dstr.at[wid], dst_v)
    for h in range(NCHUNK // HALF):
        pltpu.sync_copy(srcr.at[wid, pl.ds(h * HALF, HALF)], src_v)
        if h == 0:
            plsc.subcore_barrier()
        pltpu.async_copy(ts.at[src_v.at[0]], rows0, sem0)

        def pair(j, _):
            c = h * HALF + 2 * j
            pltpu.make_async_copy(ts.at[src_v.at[2 * j]], rows0, sem0).wait()
            pltpu.async_copy(ts.at[src_v.at[2 * j + 1]], rows1, sem1)
            pltpu.sync_copy(rows0, acc_sh.at[dst_v.at[c]], add=True)
            pltpu.make_async_copy(ts.at[src_v.at[2 * j + 1]], rows1,
                                  sem1).wait()

            @pl.when(j < HALF // 2 - 1)
            def _():
                pltpu.async_copy(ts.at[src_v.at[2 * j + 2]], rows0, sem0)
            pltpu.sync_copy(rows1, acc_sh.at[dst_v.at[c + 1]], add=True)
            return 0
        lax.fori_loop(0, HALF // 2, pair, 0)
    plsc.subcore_barrier()
    pltpu.sync_copy(acc_sh.at[pl.ds(sid * OUT_PT, OUT_PT)],
                    out.at[cid, pl.ds(sid * OUT_PT, OUT_PT)])

    @pl.when(sid == NS - 1)
    def _():
        pltpu.sync_copy(acc_sh.at[pl.ds(TAIL_OFF, TAIL)],
                        out.at[cid, pl.ds(TAIL_OFF, TAIL)])


@functools.lru_cache(maxsize=None)
def _mp_kernel():
    mesh = plsc.VectorSubcoreMesh(core_axis_name="c", subcore_axis_name="s")
    return pl.kernel(
        _mp_body,
        out_type=jax.ShapeDtypeStruct((NC, N, H), jnp.float32),
        mesh=mesh,
        scratch_types=[
            pltpu.VMEM((HALF, CH), jnp.int32),
            pltpu.VMEM((NCHUNK, CH), jnp.int32),
            pltpu.VMEM((CH, H), jnp.float32),
            pltpu.VMEM((CH, H), jnp.float32),
            pltpu.VMEM_SHARED((NPAD, H), jnp.float32),
            pltpu.SemaphoreType.DMA,
            pltpu.SemaphoreType.DMA,
        ],
    )


def _ln_relu(h, g, b):
    mu = jnp.mean(h, axis=-1, keepdims=True)
    d = h - mu
    var = jnp.mean(d * d, axis=-1, keepdims=True)
    return jnp.maximum(d * lax.rsqrt(var + 1e-5) * g + b, 0.0)


def _pre_body(x_ref, wp_ref, bp_ref, gp_ref, bep_ref, o_ref):
    h = jnp.dot(x_ref[...], wp_ref[...],
                preferred_element_type=jnp.float32) + bp_ref[...]
    o_ref[...] = _ln_relu(h, gp_ref[...], bep_ref[...])


def _dinv_of(degp_blk):
    return lax.rsqrt(degp_blk[0, :, :1] + degp_blk[1, :, :1] + 1.0)


def _ts_body(h_ref, degp_ref, w_ref, o_ref):
    dinv = _dinv_of(degp_ref[...])
    t = jnp.dot(h_ref[...], w_ref[...], preferred_element_type=jnp.float32)
    o_ref[...] = t * dinv


def _mid_body(acc_ref, tsp_ref, degp_ref, b_ref, g_ref, be_ref, w_ref, o_ref):
    dinv = _dinv_of(degp_ref[...])
    conv = (acc_ref[0] + acc_ref[1] + tsp_ref[...]) * dinv + b_ref[...]
    h = _ln_relu(conv, g_ref[...], be_ref[...])
    t = jnp.dot(h, w_ref[...], preferred_element_type=jnp.float32)
    o_ref[...] = t * dinv


def _read_body(acc_ref, tsp_ref, degp_ref, b_ref, g_ref, be_ref, bat_ref,
               wpost_ref, o_ref, sums, cnts):
    i = pl.program_id(0)

    @pl.when(i == 0)
    def _():
        sums[...] = jnp.zeros_like(sums)
        cnts[...] = jnp.zeros_like(cnts)

    dinv = _dinv_of(degp_ref[...])
    conv = (acc_ref[0] + acc_ref[1] + tsp_ref[...]) * dinv + b_ref[...]
    h = _ln_relu(conv, g_ref[...], be_ref[...])
    bt = bat_ref[0, 0, :]
    onehot = (lax.broadcasted_iota(jnp.int32, (G, BLK), 0)
              == bt[None, :]).astype(jnp.float32)
    sums[...] += jnp.dot(onehot, h, preferred_element_type=jnp.float32)
    cnts[...] = cnts[...] + jnp.sum(onehot, axis=1, keepdims=True)

    @pl.when(i == NBLK - 1)
    def _():
        emb = sums[...] / jnp.maximum(cnts[...], 1.0)
        logits = jnp.dot(emb, wpost_ref[...],
                         preferred_element_type=jnp.float32)
        col = lax.broadcasted_iota(jnp.int32, (G, H), 1)
        lp = jnp.where(col < C, logits, -1e30)
        m = jnp.max(lp, axis=1, keepdims=True)
        z = lp - m
        lse = jnp.log(jnp.sum(jnp.exp(z), axis=1, keepdims=True))
        o_ref[...] = z - lse


def _row_spec(bs):
    return pl.BlockSpec(bs, lambda i: (i, 0))


def _full_spec(bs):
    return pl.BlockSpec(bs, lambda i: tuple(0 for _ in bs))


def kernel(x, edge_index, batch, W_pre, b_pre, g_pre, be_pre, W1, b1, g1,
           be1, W2, b2, g2, be2, W_post):
    f32 = jnp.float32
    npd = jnp.arange(EPAD - E, dtype=jnp.int32)
    ei = jnp.concatenate(
        [edge_index, jnp.stack([npd % N, N + npd % (NPAD - N)])], axis=1)
    srcr = ei[0].reshape(NW, NCHUNK, CH)
    dstr = ei[1].reshape(NW, NCHUNK, CH)
    batch3 = batch.reshape(NBLK, 1, BLK)
    b_pre2, g_pre2, be_pre2 = (b_pre.reshape(1, H), g_pre.reshape(1, H),
                               be_pre.reshape(1, H))
    b12, g12, be12 = b1.reshape(1, H), g1.reshape(1, H), be1.reshape(1, H)
    b22, g22, be22 = b2.reshape(1, H), g2.reshape(1, H), be2.reshape(1, H)
    wpost_pad = jnp.pad(W_post, ((0, 0), (0, H - C)))

    degp = _deg_kernel()(dstr)

    h0 = pl.pallas_call(
        _pre_body,
        grid=(NBLK,),
        in_specs=[_row_spec((BLK, D)), _full_spec((D, H)), _full_spec((1, H)),
                  _full_spec((1, H)), _full_spec((1, H))],
        out_specs=_row_spec((BLK, H)),
        out_shape=jax.ShapeDtypeStruct((N, H), f32),
    )(x, W_pre, b_pre2, g_pre2, be_pre2)

    acc_spec = pl.BlockSpec((NC, BLK, H), lambda i: (0, i, 0))
    deg_spec = acc_spec

    ts1 = pl.pallas_call(
        _ts_body,
        grid=(NBLK,),
        in_specs=[_row_spec((BLK, H)), deg_spec, _full_spec((H, H))],
        out_specs=_row_spec((BLK, H)),
        out_shape=jax.ShapeDtypeStruct((N, H), f32),
    )(h0, degp, W1)

    accs1 = _mp_kernel()(ts1, srcr, dstr)

    ts2 = pl.pallas_call(
        _mid_body,
        grid=(NBLK,),
        in_specs=[acc_spec, _row_spec((BLK, H)), deg_spec,
                  _full_spec((1, H)), _full_spec((1, H)), _full_spec((1, H)),
                  _full_spec((H, H))],
        out_specs=_row_spec((BLK, H)),
        out_shape=jax.ShapeDtypeStruct((N, H), f32),
    )(accs1, ts1, degp, b12, g12, be12, W2)

    accs2 = _mp_kernel()(ts2, srcr, dstr)

    logits = pl.pallas_call(
        _read_body,
        grid=(NBLK,),
        in_specs=[acc_spec, _row_spec((BLK, H)), deg_spec,
                  _full_spec((1, H)), _full_spec((1, H)), _full_spec((1, H)),
                  pl.BlockSpec((1, 1, BLK), lambda i: (i, 0, 0)),
                  _full_spec((H, H))],
        out_specs=_full_spec((G, H)),
        out_shape=jax.ShapeDtypeStruct((G, H), f32),
        scratch_shapes=[pltpu.VMEM((G, H), f32), pltpu.VMEM((G, H), f32)],
    )(accs2, ts2, degp, b22, g22, be22, batch3, wpost_pad)

    return logits[:, :C]
